# baseline (speedup 1.0000x reference)
import functools

import jax
import jax.numpy as jnp
from jax.experimental import pallas as pl
from jax.experimental.pallas import tpu as pltpu

_R2 = 0.15 * 0.15


def _body(q_ref, latT_ref, fy_ref, w0_ref, b0_ref, w1_ref, b1_ref,
          p0_ref, pb0_ref, p1_ref, pb1_ref, out_ref,
          res_ref, sumf_ref, cnt_ref, *, qb_size, lc_size):
    lc = pl.program_id(2)
    nlc = pl.num_programs(2)

    @pl.when(lc == 0)
    def _():
        res_ref[...] = jnp.zeros_like(res_ref)
        sumf_ref[...] = jnp.zeros_like(sumf_ref)
        cnt_ref[...] = jnp.zeros_like(cnt_ref)

    q = q_ref[0]
    latT = latT_ref[...]

    ql = jax.lax.dot_general(q, latT, (((1,), (0,)), ((), ())),
                             precision=jax.lax.Precision.HIGHEST,
                             preferred_element_type=jnp.float32)
    qn = jnp.sum(q * q, axis=1, keepdims=True)
    ln = jnp.sum(latT * latT, axis=0, keepdims=True)
    d2 = (qn + ln) - 2.0 * ql
    mask = d2 <= _R2

    w0 = w0_ref[...]
    a_mat = w0[:3]
    b_mat = w0[3:]
    bq = jax.lax.dot_general(q, b_mat, (((1,), (0,)), ((), ())),
                             preferred_element_type=jnp.float32)
    aT = jax.lax.dot_general(a_mat, latT, (((0,), (0,)), ((), ())),
                             preferred_element_type=jnp.float32)
    aT = aT + b0_ref[...]
    h = (bq.astype(jnp.bfloat16)[:, :, None]
         + aT.astype(jnp.bfloat16)[None, :, :])
    h = jax.nn.gelu(h)
    mh = jnp.where(mask[:, None, :], h, jnp.bfloat16(0.0))

    fy = fy_ref[0]
    mh2 = mh.reshape(qb_size * 32, lc_size)
    res_ref[...] += jax.lax.dot_general(
        mh2, fy, (((1,), (0,)), ((), ())),
        preferred_element_type=jnp.float32).reshape(qb_size, 32, 128)
    sumf_ref[...] += jax.lax.dot_general(
        mask.astype(jnp.bfloat16), fy, (((1,), (0,)), ((), ())),
        preferred_element_type=jnp.float32)
    cnt_ref[...] += jnp.sum(mask.astype(jnp.float32), axis=1, keepdims=True)

    @pl.when(lc == nlc - 1)
    def _():
        w1 = w1_ref[...]
        agg = jnp.sum(res_ref[...] * w1[None, :, :], axis=1)
        agg = agg + b1_ref[...] * sumf_ref[...]
        agg = agg / jnp.maximum(cnt_ref[...], 1.0)
        h2 = jax.lax.dot_general(agg, p0_ref[...], (((1,), (0,)), ((), ())),
                                 preferred_element_type=jnp.float32)
        h2 = jax.nn.gelu(h2 + pb0_ref[...])
        out = jax.lax.dot_general(h2, p1_ref[...], (((1,), (0,)), ((), ())),
                                  preferred_element_type=jnp.float32)
        out_ref[0] = out + pb1_ref[...]


def kernel(latent_tokens_coord, rndata, query_coord, W0, b0, W1, b1,
           P0, pb0, P1, pb1):
    B, Nq, cd = query_coord.shape
    Nl, _ = latent_tokens_coord.shape
    ch = rndata.shape[-1]

    QB = 512
    LC = 256
    nqp = -(-Nq // QB) * QB
    nqb = nqp // QB
    nlc = Nl // LC
    qpad = jnp.concatenate(
        [query_coord, jnp.full((B, nqp - Nq, cd), 2.0, jnp.float32)], axis=1)

    latT = latent_tokens_coord.T
    fy = rndata.astype(jnp.bfloat16)
    b0c = b0.reshape(-1, 1)
    b1c = b1.reshape(1, -1)
    pb0c = pb0.reshape(1, -1)
    pb1c = pb1.reshape(1, -1)

    body = functools.partial(_body, qb_size=QB, lc_size=LC)

    out = pl.pallas_call(
        body,
        grid=(B, nqb, nlc),
        in_specs=[
            pl.BlockSpec((1, QB, cd), lambda b, qb, lc: (b, qb, 0)),
            pl.BlockSpec((cd, LC), lambda b, qb, lc: (0, lc)),
            pl.BlockSpec((1, LC, ch), lambda b, qb, lc: (b, lc, 0)),
            pl.BlockSpec(W0.shape, lambda b, qb, lc: (0, 0)),
            pl.BlockSpec((b0.shape[0], 1), lambda b, qb, lc: (0, 0)),
            pl.BlockSpec(W1.shape, lambda b, qb, lc: (0, 0)),
            pl.BlockSpec((1, b1.shape[0]), lambda b, qb, lc: (0, 0)),
            pl.BlockSpec(P0.shape, lambda b, qb, lc: (0, 0)),
            pl.BlockSpec((1, pb0.shape[0]), lambda b, qb, lc: (0, 0)),
            pl.BlockSpec(P1.shape, lambda b, qb, lc: (0, 0)),
            pl.BlockSpec((1, pb1.shape[0]), lambda b, qb, lc: (0, 0)),
        ],
        out_specs=pl.BlockSpec((1, QB, ch), lambda b, qb, lc: (b, qb, 0)),
        out_shape=jax.ShapeDtypeStruct((B, nqp, ch), jnp.float32),
        scratch_shapes=[
            pltpu.VMEM((QB, 32, ch), jnp.float32),
            pltpu.VMEM((QB, ch), jnp.float32),
            pltpu.VMEM((QB, 1), jnp.float32),
        ],
        compiler_params=pltpu.CompilerParams(
            dimension_semantics=("parallel", "parallel", "arbitrary"),
        ),
    )(qpad, latT, fy, W0, b0c, W1, b1c, P0, pb0c, P1, pb1c)
    return out[:, :Nq]


# speedup vs baseline: 1.2247x; 1.2247x over previous
import functools

import jax
import jax.numpy as jnp
from jax.experimental import pallas as pl
from jax.experimental.pallas import tpu as pltpu

_R2 = 0.15 * 0.15


def _body(q_ref, latT_ref, fy_ref, w0_ref, b0_ref, w1_ref, b1_ref,
          p0_ref, pb0_ref, p1_ref, pb1_ref, out_ref,
          res_ref, sumf_ref, cnt_ref, *, qb_size, lc_size):
    lc = pl.program_id(2)
    nlc = pl.num_programs(2)

    @pl.when(lc == 0)
    def _():
        res_ref[...] = jnp.zeros_like(res_ref)
        sumf_ref[...] = jnp.zeros_like(sumf_ref)
        cnt_ref[...] = jnp.zeros_like(cnt_ref)

    q = q_ref[0]
    latT = latT_ref[...]

    ql = jax.lax.dot_general(q, latT, (((1,), (0,)), ((), ())),
                             precision=jax.lax.Precision.HIGHEST,
                             preferred_element_type=jnp.float32)
    qn = jnp.sum(q * q, axis=1, keepdims=True)
    ln = jnp.sum(latT * latT, axis=0, keepdims=True)
    d2 = (qn + ln) - 2.0 * ql
    mask = d2 <= _R2

    w0 = w0_ref[...]
    a_mat = w0[:3]
    b_mat = w0[3:]
    bq = jax.lax.dot_general(q, b_mat, (((1,), (0,)), ((), ())),
                             preferred_element_type=jnp.float32)
    aT = jax.lax.dot_general(a_mat, latT, (((0,), (0,)), ((), ())),
                             preferred_element_type=jnp.float32)
    aT = aT + b0_ref[...]
    h = (bq.astype(jnp.bfloat16)[:, :, None]
         + aT.astype(jnp.bfloat16)[None, :, :])
    h = jax.nn.gelu(h)
    mh = jnp.where(mask[:, None, :], h, jnp.bfloat16(0.0))

    fy = fy_ref[0]
    mh2 = mh.reshape(qb_size * 32, lc_size)
    res_ref[...] += jax.lax.dot_general(
        mh2, fy, (((1,), (0,)), ((), ())),
        preferred_element_type=jnp.float32).reshape(qb_size, 32, 128)
    sumf_ref[...] += jax.lax.dot_general(
        mask.astype(jnp.bfloat16), fy, (((1,), (0,)), ((), ())),
        preferred_element_type=jnp.float32)
    cnt_ref[...] += jnp.sum(mask.astype(jnp.float32), axis=1, keepdims=True)

    @pl.when(lc == nlc - 1)
    def _():
        w1 = w1_ref[...]
        agg = jnp.sum(res_ref[...] * w1[None, :, :], axis=1)
        agg = agg + b1_ref[...] * sumf_ref[...]
        agg = agg / jnp.maximum(cnt_ref[...], 1.0)
        h2 = jax.lax.dot_general(agg, p0_ref[...], (((1,), (0,)), ((), ())),
                                 preferred_element_type=jnp.float32)
        h2 = jax.nn.gelu(h2 + pb0_ref[...])
        out = jax.lax.dot_general(h2, p1_ref[...], (((1,), (0,)), ((), ())),
                                  preferred_element_type=jnp.float32)
        out_ref[0] = out + pb1_ref[...]


def kernel(latent_tokens_coord, rndata, query_coord, W0, b0, W1, b1,
           P0, pb0, P1, pb1):
    B, Nq, cd = query_coord.shape
    Nl, _ = latent_tokens_coord.shape
    ch = rndata.shape[-1]

    QB = 400
    LC = 512
    nqb = Nq // QB
    nlc = Nl // LC

    latT = latent_tokens_coord.T
    fy = rndata.astype(jnp.bfloat16)
    b0c = b0.reshape(-1, 1)
    b1c = b1.reshape(1, -1)
    pb0c = pb0.reshape(1, -1)
    pb1c = pb1.reshape(1, -1)

    body = functools.partial(_body, qb_size=QB, lc_size=LC)

    out = pl.pallas_call(
        body,
        grid=(B, nqb, nlc),
        in_specs=[
            pl.BlockSpec((1, QB, cd), lambda b, qb, lc: (b, qb, 0)),
            pl.BlockSpec((cd, LC), lambda b, qb, lc: (0, lc)),
            pl.BlockSpec((1, LC, ch), lambda b, qb, lc: (b, lc, 0)),
            pl.BlockSpec(W0.shape, lambda b, qb, lc: (0, 0)),
            pl.BlockSpec((b0.shape[0], 1), lambda b, qb, lc: (0, 0)),
            pl.BlockSpec(W1.shape, lambda b, qb, lc: (0, 0)),
            pl.BlockSpec((1, b1.shape[0]), lambda b, qb, lc: (0, 0)),
            pl.BlockSpec(P0.shape, lambda b, qb, lc: (0, 0)),
            pl.BlockSpec((1, pb0.shape[0]), lambda b, qb, lc: (0, 0)),
            pl.BlockSpec(P1.shape, lambda b, qb, lc: (0, 0)),
            pl.BlockSpec((1, pb1.shape[0]), lambda b, qb, lc: (0, 0)),
        ],
        out_specs=pl.BlockSpec((1, QB, ch), lambda b, qb, lc: (b, qb, 0)),
        out_shape=jax.ShapeDtypeStruct((B, Nq, ch), jnp.float32),
        scratch_shapes=[
            pltpu.VMEM((QB, 32, ch), jnp.float32),
            pltpu.VMEM((QB, ch), jnp.float32),
            pltpu.VMEM((QB, 1), jnp.float32),
        ],
        compiler_params=pltpu.CompilerParams(
            dimension_semantics=("parallel", "parallel", "arbitrary"),
        ),
    )(query_coord, latT, fy, W0, b0c, W1, b1c, P0, pb0c, P1, pb1c)
    return out


# speedup vs baseline: 1.2652x; 1.0331x over previous
import functools

import jax
import jax.numpy as jnp
from jax.experimental import pallas as pl
from jax.experimental.pallas import tpu as pltpu

_R2 = 0.15 * 0.15


def _body(q_ref, latT_ref, fy_ref, w0_ref, b0_ref, w1_ref, b1_ref,
          p0_ref, pb0_ref, p1_ref, pb1_ref, out_ref,
          res_ref, sumf_ref, cnt_ref, *, qb_size, lc_size):
    lc = pl.program_id(2)
    nlc = pl.num_programs(2)

    @pl.when(lc == 0)
    def _():
        res_ref[...] = jnp.zeros_like(res_ref)
        sumf_ref[...] = jnp.zeros_like(sumf_ref)
        cnt_ref[...] = jnp.zeros_like(cnt_ref)

    q = q_ref[0]
    latT = latT_ref[...]

    ql = jax.lax.dot_general(q, latT, (((1,), (0,)), ((), ())),
                             precision=jax.lax.Precision.HIGHEST,
                             preferred_element_type=jnp.float32)
    qn = jnp.sum(q * q, axis=1, keepdims=True)
    ln = jnp.sum(latT * latT, axis=0, keepdims=True)
    d2 = (qn + ln) - 2.0 * ql
    mask = d2 <= _R2

    w0 = w0_ref[...]
    a_mat = w0[:3]
    b_mat = w0[3:]
    bq = jax.lax.dot_general(q, b_mat, (((1,), (0,)), ((), ())),
                             preferred_element_type=jnp.float32)
    aT = jax.lax.dot_general(a_mat, latT, (((0,), (0,)), ((), ())),
                             preferred_element_type=jnp.float32)
    aT = aT + b0_ref[...]
    h = (bq.astype(jnp.bfloat16)[:, :, None]
         + aT.astype(jnp.bfloat16)[None, :, :])
    h = jax.nn.gelu(h)
    mh = jnp.where(mask[:, None, :], h, jnp.bfloat16(0.0))

    fy = fy_ref[0]
    mh2 = mh.reshape(qb_size * 32, lc_size)
    res_ref[...] += jax.lax.dot_general(
        mh2, fy, (((1,), (0,)), ((), ())),
        preferred_element_type=jnp.float32).reshape(qb_size, 32, 128)
    sumf_ref[...] += jax.lax.dot_general(
        mask.astype(jnp.bfloat16), fy, (((1,), (0,)), ((), ())),
        preferred_element_type=jnp.float32)
    cnt_ref[...] += jnp.sum(mask.astype(jnp.float32), axis=1, keepdims=True)

    @pl.when(lc == nlc - 1)
    def _():
        w1 = w1_ref[...]
        agg = jnp.sum(res_ref[...] * w1[None, :, :], axis=1)
        agg = agg + b1_ref[...] * sumf_ref[...]
        agg = agg / jnp.maximum(cnt_ref[...], 1.0)
        h2 = jax.lax.dot_general(agg, p0_ref[...], (((1,), (0,)), ((), ())),
                                 preferred_element_type=jnp.float32)
        h2 = jax.nn.gelu(h2 + pb0_ref[...])
        out = jax.lax.dot_general(h2, p1_ref[...], (((1,), (0,)), ((), ())),
                                  preferred_element_type=jnp.float32)
        out_ref[0] = out + pb1_ref[...]


def kernel(latent_tokens_coord, rndata, query_coord, W0, b0, W1, b1,
           P0, pb0, P1, pb1):
    B, Nq, cd = query_coord.shape
    Nl, _ = latent_tokens_coord.shape
    ch = rndata.shape[-1]

    QB = 400
    LC = 1024
    nqb = Nq // QB
    nlc = Nl // LC

    latT = latent_tokens_coord.T
    fy = rndata.astype(jnp.bfloat16)
    b0c = b0.reshape(-1, 1)
    b1c = b1.reshape(1, -1)
    pb0c = pb0.reshape(1, -1)
    pb1c = pb1.reshape(1, -1)

    body = functools.partial(_body, qb_size=QB, lc_size=LC)

    out = pl.pallas_call(
        body,
        grid=(B, nqb, nlc),
        in_specs=[
            pl.BlockSpec((1, QB, cd), lambda b, qb, lc: (b, qb, 0)),
            pl.BlockSpec((cd, LC), lambda b, qb, lc: (0, lc)),
            pl.BlockSpec((1, LC, ch), lambda b, qb, lc: (b, lc, 0)),
            pl.BlockSpec(W0.shape, lambda b, qb, lc: (0, 0)),
            pl.BlockSpec((b0.shape[0], 1), lambda b, qb, lc: (0, 0)),
            pl.BlockSpec(W1.shape, lambda b, qb, lc: (0, 0)),
            pl.BlockSpec((1, b1.shape[0]), lambda b, qb, lc: (0, 0)),
            pl.BlockSpec(P0.shape, lambda b, qb, lc: (0, 0)),
            pl.BlockSpec((1, pb0.shape[0]), lambda b, qb, lc: (0, 0)),
            pl.BlockSpec(P1.shape, lambda b, qb, lc: (0, 0)),
            pl.BlockSpec((1, pb1.shape[0]), lambda b, qb, lc: (0, 0)),
        ],
        out_specs=pl.BlockSpec((1, QB, ch), lambda b, qb, lc: (b, qb, 0)),
        out_shape=jax.ShapeDtypeStruct((B, Nq, ch), jnp.float32),
        scratch_shapes=[
            pltpu.VMEM((QB, 32, ch), jnp.float32),
            pltpu.VMEM((QB, ch), jnp.float32),
            pltpu.VMEM((QB, 1), jnp.float32),
        ],
        compiler_params=pltpu.CompilerParams(
            dimension_semantics=("parallel", "parallel", "arbitrary"),
        ),
    )(query_coord, latT, fy, W0, b0c, W1, b1c, P0, pb0c, P1, pb1c)
    return out
